# baseline (speedup 1.0000x reference)
import jax
import jax.numpy as jnp
from jax import lax
from jax.experimental import pallas as pl
from jax.experimental.pallas import tpu as pltpu
from jax.experimental.pallas import tpu_sc as plsc

_VOCAB = 100000
_D = 64
_B = 16384
_H = 50

_NC = 2
_NS = 16
_NW = _NC * _NS
_BPW = _B // _NW


def _bag_body(ids_t_hbm, table_hbm, out_hbm, idx_v, acc_v, acct_v, sem):
    wid = lax.axis_index("s") * _NC + lax.axis_index("c")
    base = wid * _BPW

    pltpu.sync_copy(ids_t_hbm.at[:, pl.ds(base, _BPW)], idx_v)

    pltpu.async_copy(table_hbm.at[idx_v.at[0]], acc_v, sem).wait()

    def _fire(j, carry):
        pltpu.async_copy(table_hbm.at[idx_v.at[j]], acc_v, sem, add=True)
        return carry

    lax.fori_loop(1, _H, _fire, 0)

    def _drain(j, carry):
        pltpu.make_async_copy(table_hbm.at[idx_v.at[0]], acc_v, sem).wait()
        return carry

    lax.fori_loop(1, _H, _drain, 0)

    lanes = lax.iota(jnp.int32, 16)

    def _xpose(g, carry):
        rows = lanes + g * 16
        for d in range(_D):
            v = plsc.load_gather(acc_v, [rows, jnp.full((16,), d, jnp.int32)])
            acct_v[d, pl.ds(g * 16, 16)] = v
        return carry

    lax.fori_loop(0, _BPW // 16, _xpose, 0)
    pltpu.sync_copy(acct_v, out_hbm.at[:, pl.ds(base, _BPW)])


_bag = pl.kernel(
    _bag_body,
    out_type=jax.ShapeDtypeStruct((_D, _B), jnp.float32),
    mesh=plsc.VectorSubcoreMesh(core_axis_name="c", subcore_axis_name="s"),
    scratch_types=[
        pltpu.VMEM((_H, _BPW), jnp.int32),
        pltpu.VMEM((_BPW, _D), jnp.float32),
        pltpu.VMEM((_D, _BPW), jnp.float32),
        pltpu.SemaphoreType.DMA,
    ],
    compiler_params=pltpu.CompilerParams(
        use_tc_tiling_on_sc=False, needs_layout_passes=False),
)


def kernel(ingredient_ids, embedding_table):
    ids_t = jnp.transpose(ingredient_ids.astype(jnp.int32))
    return jnp.transpose(_bag(ids_t, embedding_table))


# speedup vs baseline: 1.0537x; 1.0537x over previous
import jax
import jax.numpy as jnp
from jax import lax
from jax.experimental import pallas as pl
from jax.experimental.pallas import tpu as pltpu
from jax.experimental.pallas import tpu_sc as plsc

_VOCAB = 100000
_D = 64
_B = 16384
_H = 50

_NC = 2
_NS = 16
_NW = _NC * _NS
_BPW = _B // _NW


def _bag_body(zeros_hbm, ids_t_hbm, table_hbm, out_hbm, idx_v, acc_v, sem):
    wid = lax.axis_index("s") * _NC + lax.axis_index("c")
    base = wid * _BPW

    zfill = pltpu.async_copy(zeros_hbm, acc_v, sem)
    pltpu.sync_copy(ids_t_hbm.at[:, pl.ds(base, _BPW)], idx_v)
    zfill.wait()

    def _fire(j, carry):
        pltpu.async_copy(table_hbm.at[idx_v.at[j]], acc_v, sem, add=True)
        return carry

    lax.fori_loop(0, _H, _fire, 0)

    def _drain(j, carry):
        pltpu.make_async_copy(table_hbm.at[idx_v.at[0]], acc_v, sem).wait()
        return carry

    lax.fori_loop(0, _H, _drain, 0)

    pltpu.sync_copy(acc_v, out_hbm.at[pl.ds(base, _BPW)])


_bag = pl.kernel(
    _bag_body,
    out_type=jax.ShapeDtypeStruct((_B, _D), jnp.float32),
    mesh=plsc.VectorSubcoreMesh(core_axis_name="c", subcore_axis_name="s"),
    scratch_types=[
        pltpu.VMEM((_H, _BPW), jnp.int32),
        pltpu.VMEM((_BPW, _D), jnp.float32),
        pltpu.SemaphoreType.DMA,
    ],
    compiler_params=pltpu.CompilerParams(use_tc_tiling_on_sc=False),
)


def kernel(ingredient_ids, embedding_table):
    ids_t = jnp.transpose(ingredient_ids.astype(jnp.int32))
    zeros = jnp.zeros((_BPW, _D), jnp.float32)
    return _bag(zeros, ids_t, embedding_table)


# speedup vs baseline: 1.0895x; 1.0340x over previous
import jax
import jax.numpy as jnp
from jax import lax
from jax.experimental import pallas as pl
from jax.experimental.pallas import tpu as pltpu
from jax.experimental.pallas import tpu_sc as plsc

_VOCAB = 100000
_D = 64
_B = 16384
_H = 50

_NC = 2
_NS = 16
_NW = _NC * _NS
_BPW = _B // _NW


def _bag_body(ids_t_hbm, table_hbm, out_hbm, idx_v, acc_v, sem):
    wid = lax.axis_index("s") * _NC + lax.axis_index("c")
    base = wid * _BPW

    pltpu.sync_copy(ids_t_hbm.at[:, pl.ds(base, _BPW)], idx_v)

    pltpu.async_copy(table_hbm.at[idx_v.at[0]], acc_v, sem).wait()

    def _fire(j, carry):
        pltpu.async_copy(table_hbm.at[idx_v.at[j]], acc_v, sem, add=True)
        return carry

    lax.fori_loop(1, _H, _fire, 0)

    def _drain(j, carry):
        pltpu.make_async_copy(table_hbm.at[idx_v.at[0]], acc_v, sem).wait()
        return carry

    lax.fori_loop(1, _H, _drain, 0)

    pltpu.sync_copy(acc_v, out_hbm.at[pl.ds(base, _BPW)])


_bag = pl.kernel(
    _bag_body,
    out_type=jax.ShapeDtypeStruct((_B, _D), jnp.float32),
    mesh=plsc.VectorSubcoreMesh(core_axis_name="c", subcore_axis_name="s"),
    scratch_types=[
        pltpu.VMEM((_H, _BPW), jnp.int32),
        pltpu.VMEM((_BPW, _D), jnp.float32),
        pltpu.SemaphoreType.DMA,
    ],
    compiler_params=pltpu.CompilerParams(use_tc_tiling_on_sc=False),
)


def kernel(ingredient_ids, embedding_table):
    ids_t = jnp.transpose(ingredient_ids.astype(jnp.int32))
    return _bag(ids_t, embedding_table)
